# Initial kernel scaffold; baseline (speedup 1.0000x reference)
import functools

import numpy as np
import jax
import jax.numpy as jnp
from jax import lax
from jax.experimental import pallas as pl
from jax.experimental.pallas import tpu as pltpu
from jax.experimental.pallas import tpu_sc as plsc

GRID_RES = (32, 64, 128, 256)
NUM_LOD = len(GRID_RES)
NC, NS = 2, 16
NW = NC * NS
C = 2048
ROWS = C // 128
GROUPS = 128 // 16
NCHUNK = 16
W = C * NCHUNK
N_PAD = NW * W

_mesh = plsc.VectorSubcoreMesh(core_axis_name="c", subcore_axis_name="s")


@functools.partial(
    pl.kernel,
    mesh=_mesh,
    out_type=jax.ShapeDtypeStruct((N_PAD // 128, 128), jnp.float32),
    scratch_types=[
        pltpu.VMEM((3 * C,), jnp.float32),
        pltpu.VMEM((NUM_LOD, ROWS, 128), jnp.int32),
        pltpu.VMEM((NUM_LOD, ROWS, 128), jnp.float32),
        pltpu.VMEM((ROWS, 128), jnp.float32),
        pltpu.SemaphoreType.DMA,
    ],
)
def _grid_gather(pts_hbm, cb0_hbm, cb1_hbm, cb2_hbm, cb3_hbm, out_hbm,
                 pts_v, idx_v, feat_v, out_v, sem):
    cbs = (cb0_hbm, cb1_hbm, cb2_hbm, cb3_hbm)
    wid = lax.axis_index("s") * NC + lax.axis_index("c")
    lanes = lax.iota(jnp.int32, 16)

    def chunk_body(t, carry):
        base = wid * W + t * C
        pltpu.sync_copy(pts_hbm.at[pl.ds(base * 3, 3 * C)], pts_v)

        def row_body(r, carry2):
            for u in range(GROUPS):
                i3 = lanes * 3 + (r * 128 + u * 16) * 3
                hx = plsc.load_gather(pts_v, [i3]) * 0.5 + 0.5
                hy = plsc.load_gather(pts_v, [i3 + 1]) * 0.5 + 0.5
                hz = plsc.load_gather(pts_v, [i3 + 2]) * 0.5 + 0.5
                for l, res in enumerate(GRID_RES):
                    s = np.float32(res - 1)
                    ix = (hx * s).astype(jnp.int32)
                    iy = (hy * s).astype(jnp.int32)
                    iz = (hz * s).astype(jnp.int32)
                    idx_v[l, r, pl.ds(u * 16, 16)] = ix + iy * res + iz * (res * res)
            return carry2

        lax.fori_loop(0, ROWS, row_body, 0)

        copies = [pltpu.async_copy(cb.at[idx_v.at[l]], feat_v.at[l], sem)
                  for l, cb in enumerate(cbs)]
        for cp in copies:
            cp.wait()

        def sum_body(r, carry2):
            for u in range(GROUPS):
                acc = feat_v[0, r, pl.ds(u * 16, 16)]
                for l in range(1, NUM_LOD):
                    acc = acc + feat_v[l, r, pl.ds(u * 16, 16)]
                out_v[r, pl.ds(u * 16, 16)] = acc
            return carry2

        lax.fori_loop(0, ROWS, sum_body, 0)
        pltpu.sync_copy(out_v, out_hbm.at[pl.ds(base // 128, ROWS)])
        return carry

    lax.fori_loop(0, NCHUNK, chunk_body, 0)


def kernel(pts, cb0, cb1, cb2, cb3):
    n = pts.shape[0]
    flat = pts.reshape(-1)
    flat = jnp.concatenate(
        [flat, jnp.zeros((N_PAD * 3 - flat.shape[0],), jnp.float32)])
    out = _grid_gather(flat, cb0.reshape(-1), cb1.reshape(-1),
                       cb2.reshape(-1), cb3.reshape(-1))
    return out.reshape(-1)[:n][:, None]


# speedup vs baseline: 3.7167x; 3.7167x over previous
import functools

import numpy as np
import jax
import jax.numpy as jnp
from jax import lax
from jax.experimental import pallas as pl
from jax.experimental.pallas import tpu as pltpu
from jax.experimental.pallas import tpu_sc as plsc

GRID_RES = (32, 64, 128, 256)
NUM_LOD = len(GRID_RES)
NC, NS = 2, 16
NW = NC * NS
C = 2048
ROWS = C // 128
NGRP = C // 16
NCHUNK = 16
W = C * NCHUNK
N_PAD = NW * W

_mesh = plsc.VectorSubcoreMesh(core_axis_name="c", subcore_axis_name="s")


@functools.partial(
    pl.kernel,
    mesh=_mesh,
    out_type=jax.ShapeDtypeStruct((N_PAD,), jnp.float32),
    scratch_types=[
        pltpu.VMEM((3 * C,), jnp.float32),
        pltpu.VMEM((NUM_LOD, ROWS, 128), jnp.int32),
        pltpu.VMEM((NUM_LOD, ROWS, 128), jnp.float32),
        pltpu.VMEM((C,), jnp.float32),
        pltpu.SemaphoreType.DMA,
    ],
    compiler_params=pltpu.CompilerParams(needs_layout_passes=False),
)
def _grid_gather(pts_hbm, cb0_hbm, cb1_hbm, cb2_hbm, cb3_hbm, out_hbm,
                 pts_v, idx_v, feat_v, out_v, sem):
    cbs = (cb0_hbm, cb1_hbm, cb2_hbm, cb3_hbm)
    wid = lax.axis_index("s") * NC + lax.axis_index("c")
    lanes = lax.iota(jnp.int32, 16)

    def chunk_body(t, carry):
        base = wid * W + t * C
        pltpu.sync_copy(pts_hbm.at[pl.ds(base * 3, 3 * C)], pts_v)

        def idx_body(r, carry2):
            for u in range(128 // 16):
                i3 = lanes * 3 + (r * 128 + u * 16) * 3
                hx = plsc.load_gather(pts_v, [i3]) * 0.5 + 0.5
                hy = plsc.load_gather(pts_v, [i3 + 1]) * 0.5 + 0.5
                hz = plsc.load_gather(pts_v, [i3 + 2]) * 0.5 + 0.5
                for l, res in enumerate(GRID_RES):
                    s = np.float32(res - 1)
                    ix = (hx * s).astype(jnp.int32)
                    iy = (hy * s).astype(jnp.int32)
                    iz = (hz * s).astype(jnp.int32)
                    idx_v[l, r, pl.ds(u * 16, 16)] = (
                        ix + iy * res + iz * (res * res))
            return carry2

        lax.fori_loop(0, ROWS, idx_body, 0)

        def gather_body(r, carry2):
            copies = [pltpu.async_copy(cb.at[idx_v.at[l, r]],
                                       feat_v.at[l, r], sem)
                      for l, cb in enumerate(cbs)]
            for cp in copies:
                cp.wait()
            return carry2

        lax.fori_loop(0, ROWS, gather_body, 0)

        def sum_body(r, carry2):
            for u in range(128 // 16):
                acc = feat_v[0, r, pl.ds(u * 16, 16)]
                for l in range(1, NUM_LOD):
                    acc = acc + feat_v[l, r, pl.ds(u * 16, 16)]
                out_v[pl.ds(r * 128 + u * 16, 16)] = acc
            return carry2

        lax.fori_loop(0, ROWS, sum_body, 0)
        pltpu.sync_copy(out_v, out_hbm.at[pl.ds(base, C)])
        return carry

    lax.fori_loop(0, NCHUNK, chunk_body, 0)


def kernel(pts, cb0, cb1, cb2, cb3):
    n = pts.shape[0]
    flat = pts.reshape(-1)
    flat = jnp.concatenate(
        [flat, jnp.zeros((N_PAD * 3 - flat.shape[0],), jnp.float32)])
    out = _grid_gather(flat, cb0.reshape(-1), cb1.reshape(-1),
                       cb2.reshape(-1), cb3.reshape(-1))
    return out[:n][:, None]


# speedup vs baseline: 3.8739x; 1.0423x over previous
import functools

import numpy as np
import jax
import jax.numpy as jnp
from jax import lax
from jax.experimental import pallas as pl
from jax.experimental.pallas import tpu as pltpu
from jax.experimental.pallas import tpu_sc as plsc

GRID_RES = (32, 64, 128, 256)
NUM_LOD = len(GRID_RES)
NC, NS = 2, 16
NW = NC * NS
C = 2048
ROWS = C // 128
NGRP = C // 16
NCHUNK = 16
W = C * NCHUNK
N_PAD = NW * W

_mesh = plsc.VectorSubcoreMesh(core_axis_name="c", subcore_axis_name="s")


@functools.partial(
    pl.kernel,
    mesh=_mesh,
    out_type=jax.ShapeDtypeStruct((N_PAD,), jnp.float32),
    scratch_types=[
        pltpu.VMEM((3 * C,), jnp.float32),
        pltpu.VMEM((NUM_LOD, ROWS, 128), jnp.int32),
        pltpu.VMEM((NUM_LOD, ROWS, 128), jnp.float32),
        pltpu.VMEM((C,), jnp.float32),
        pltpu.SemaphoreType.DMA,
    ],
    compiler_params=pltpu.CompilerParams(needs_layout_passes=False),
)
def _grid_gather(pts_hbm, cb0_hbm, cb1_hbm, cb2_hbm, cb3_hbm, out_hbm,
                 pts_v, idx_v, feat_v, out_v, sem):
    cbs = (cb0_hbm, cb1_hbm, cb2_hbm, cb3_hbm)
    wid = lax.axis_index("s") * NC + lax.axis_index("c")
    lanes = lax.iota(jnp.int32, 16)

    def chunk_body(t, carry):
        base = wid * W + t * C
        pltpu.sync_copy(pts_hbm.at[pl.ds(base * 3, 3 * C)], pts_v)

        def idx_fire(r, carry2):
            for u in range(128 // 16):
                i3 = lanes * 3 + (r * 128 + u * 16) * 3
                hx = plsc.load_gather(pts_v, [i3]) * 0.5 + 0.5
                hy = plsc.load_gather(pts_v, [i3 + 1]) * 0.5 + 0.5
                hz = plsc.load_gather(pts_v, [i3 + 2]) * 0.5 + 0.5
                for l, res in enumerate(GRID_RES):
                    s = np.float32(res - 1)
                    ix = (hx * s).astype(jnp.int32)
                    iy = (hy * s).astype(jnp.int32)
                    iz = (hz * s).astype(jnp.int32)
                    idx_v[l, r, pl.ds(u * 16, 16)] = (
                        ix + iy * res + iz * (res * res))
            for l, cb in enumerate(cbs):
                pltpu.async_copy(cb.at[idx_v.at[l, r]], feat_v.at[l, r], sem)
            return carry2

        lax.fori_loop(0, ROWS, idx_fire, 0)

        def drain_body(r, carry2):
            for l, cb in enumerate(cbs):
                pltpu.make_async_copy(cb.at[idx_v.at[l, r]],
                                      feat_v.at[l, r], sem).wait()
            return carry2

        lax.fori_loop(0, ROWS, drain_body, 0)

        def sum_body(r, carry2):
            for u in range(128 // 16):
                acc = feat_v[0, r, pl.ds(u * 16, 16)]
                for l in range(1, NUM_LOD):
                    acc = acc + feat_v[l, r, pl.ds(u * 16, 16)]
                out_v[pl.ds(r * 128 + u * 16, 16)] = acc
            return carry2

        lax.fori_loop(0, ROWS, sum_body, 0)
        pltpu.sync_copy(out_v, out_hbm.at[pl.ds(base, C)])
        return carry

    lax.fori_loop(0, NCHUNK, chunk_body, 0)


def kernel(pts, cb0, cb1, cb2, cb3):
    n = pts.shape[0]
    flat = pts.reshape(-1)
    flat = jnp.concatenate(
        [flat, jnp.zeros((N_PAD * 3 - flat.shape[0],), jnp.float32)])
    out = _grid_gather(flat, cb0.reshape(-1), cb1.reshape(-1),
                       cb2.reshape(-1), cb3.reshape(-1))
    return out[:n][:, None]
